# baseline (speedup 1.0000x reference)
import functools

import jax
import jax.numpy as jnp
from jax import lax
from jax.experimental import pallas as pl
from jax.experimental.pallas import tpu as pltpu
from jax.experimental.pallas import tpu_sc as plsc

_NUM_SC = 2
_NUM_SUBCORES = 16


def _tc_body(lens_ref, x_ref, codes_ref, W_ref, bias_ref, u_ref,
             out_ref, val_ref, idx_ref):
    b_id = pl.program_id(0)
    xb = x_ref[0]
    s = jax.nn.sigmoid(xb)
    h = jnp.tanh(
        jnp.dot(s, W_ref[...], preferred_element_type=jnp.float32)
        + bias_ref[...]
    )
    vu = jnp.sum(h * u_ref[...], axis=1, keepdims=True)
    T = xb.shape[0]
    C = xb.shape[1]
    t_iota = jax.lax.broadcasted_iota(jnp.int32, (T, 1), 0)
    vu = jnp.where(t_iota < lens_ref[b_id], vu, -1e9)
    m = jnp.max(vu, axis=0, keepdims=True)
    e = jnp.exp(vu - m)
    score = e / jnp.sum(e, axis=0, keepdims=True)

    codes = codes_ref[0]
    c_iota = jax.lax.broadcasted_iota(jnp.int32, xb.shape, 1)
    onehot = c_iota == codes
    x_g = jnp.sum(jnp.where(onehot, xb, 0.0), axis=1, keepdims=True)
    out_ref[0] = s
    val_ref[0] = jax.nn.sigmoid(x_g + score)
    idx_ref[0] = (b_id * T + t_iota) * C + codes


def _tc_part(x, lens, codes3, W, bias2, u2):
    B, T, C = x.shape
    A = W.shape[1]
    return pl.pallas_call(
        _tc_body,
        grid=(B,),
        in_specs=[
            pl.BlockSpec(memory_space=pltpu.SMEM),
            pl.BlockSpec((1, T, C), lambda i: (i, 0, 0)),
            pl.BlockSpec((1, T, 1), lambda i: (i, 0, 0)),
            pl.BlockSpec((C, A), lambda i: (0, 0)),
            pl.BlockSpec((1, A), lambda i: (0, 0)),
            pl.BlockSpec((1, A), lambda i: (0, 0)),
        ],
        out_specs=[
            pl.BlockSpec((1, T, C), lambda i: (i, 0, 0)),
            pl.BlockSpec((1, T, 1), lambda i: (i, 0, 0)),
            pl.BlockSpec((1, T, 1), lambda i: (i, 0, 0)),
        ],
        out_shape=[
            jax.ShapeDtypeStruct((B, T, C), jnp.float32),
            jax.ShapeDtypeStruct((B, T, 1), jnp.float32),
            jax.ShapeDtypeStruct((B, T, 1), jnp.int32),
        ],
        compiler_params=pltpu.CompilerParams(
            dimension_semantics=("arbitrary",),
        ),
    )(lens, x, codes3, W, bias2, u2)


def _sc_scatter_body(chunk, idx_hbm, val_hbm, out_hbm, idx_v, val_v, sem):
    wid = lax.axis_index("s") * _NUM_SC + lax.axis_index("c")
    base = wid * chunk
    pltpu.sync_copy(idx_hbm.at[pl.ds(base, chunk)], idx_v)
    pltpu.sync_copy(val_hbm.at[pl.ds(base, chunk)], val_v)
    pltpu.async_copy(val_v, out_hbm.at[idx_v], sem).wait()


def _sc_scatter(idx_flat, val_flat, out_mutref, chunk):
    mesh = plsc.VectorSubcoreMesh(
        core_axis_name="c", subcore_axis_name="s",
        num_cores=_NUM_SC, num_subcores=_NUM_SUBCORES,
    )
    body = functools.partial(_sc_scatter_body, chunk)
    run = pl.kernel(
        body,
        out_type=(),
        mesh=mesh,
        scratch_types=[
            pltpu.VMEM((chunk,), jnp.int32),
            pltpu.VMEM((chunk,), jnp.float32),
            pltpu.SemaphoreType.DMA,
        ],
    )
    run(idx_flat, val_flat, out_mutref)


def kernel(x, lens, target_codes, W, b, u):
    B, T, C = x.shape
    A = W.shape[1]
    codes3 = target_codes.reshape(B, T, 1)
    bias2 = b.reshape(1, A)
    u2 = u.reshape(1, A)
    dense, val, idx = _tc_part(x, lens, codes3, W, bias2, u2)
    out_ref = jax.new_ref(dense.reshape(B * T * C))
    _sc_scatter(idx.reshape(B * T), val.reshape(B * T), out_ref,
                (B * T) // (_NUM_SC * _NUM_SUBCORES))
    return out_ref[...].reshape(B, T, C)


# speedup vs baseline: 3.5664x; 3.5664x over previous
import jax
import jax.numpy as jnp
from jax.experimental import pallas as pl
from jax.experimental.pallas import tpu as pltpu


def _body(lens_ref, x_ref, codes_ref, W_ref, bias_ref, u_ref, out_ref):
    b_id = pl.program_id(0)
    xb = x_ref[0]
    s = jax.nn.sigmoid(xb)
    h = jnp.tanh(
        jnp.dot(s, W_ref[...], preferred_element_type=jnp.float32)
        + bias_ref[...]
    )
    vu = jnp.sum(h * u_ref[...], axis=1, keepdims=True)
    T = xb.shape[0]
    t_iota = jax.lax.broadcasted_iota(jnp.int32, (T, 1), 0)
    vu = jnp.where(t_iota < lens_ref[b_id], vu, -1e9)
    m = jnp.max(vu, axis=0, keepdims=True)
    e = jnp.exp(vu - m)
    score = e / jnp.sum(e, axis=0, keepdims=True)

    codes = codes_ref[0]
    c_iota = jax.lax.broadcasted_iota(jnp.int32, xb.shape, 1)
    onehot = c_iota == codes
    x_g = jnp.sum(jnp.where(onehot, xb, 0.0), axis=1, keepdims=True)
    val = jax.nn.sigmoid(x_g + score)
    out_ref[0] = jnp.where(onehot, val, s)


def kernel(x, lens, target_codes, W, b, u):
    B, T, C = x.shape
    A = W.shape[1]
    codes3 = target_codes.reshape(B, T, 1)
    bias2 = b.reshape(1, A)
    u2 = u.reshape(1, A)
    return pl.pallas_call(
        _body,
        grid=(B,),
        in_specs=[
            pl.BlockSpec(memory_space=pltpu.SMEM),
            pl.BlockSpec((1, T, C), lambda i: (i, 0, 0)),
            pl.BlockSpec((1, T, 1), lambda i: (i, 0, 0)),
            pl.BlockSpec((C, A), lambda i: (0, 0)),
            pl.BlockSpec((1, A), lambda i: (0, 0)),
            pl.BlockSpec((1, A), lambda i: (0, 0)),
        ],
        out_specs=pl.BlockSpec((1, T, C), lambda i: (i, 0, 0)),
        out_shape=jax.ShapeDtypeStruct((B, T, C), jnp.float32),
        compiler_params=pltpu.CompilerParams(
            dimension_semantics=("arbitrary",),
        ),
    )(lens, x, codes3, W, bias2, u2)
